# baseline (speedup 1.0000x reference)
import functools

import jax
import jax.numpy as jnp
from jax import lax
from jax.experimental import pallas as pl
from jax.experimental.pallas import tpu as pltpu
from jax.experimental.pallas import tpu_sc as plsc

_B = 4096
_DJ, _DC, _DT, _DD = 64, 64, 32, 128
_TOTAL = _DJ + _DC + _DT + _DD
_OUT = 128
_NC, _NS = 2, 16
_NW = _NC * _NS
_BPW = _B // _NW


def _sc_gather(job_id, company_id, title_id, emb_job, emb_company, emb_title):
    mesh = plsc.VectorSubcoreMesh(
        core_axis_name="c", subcore_axis_name="s",
        num_cores=_NC, num_subcores=_NS,
    )

    @functools.partial(
        pl.kernel,
        out_type=(
            jax.ShapeDtypeStruct((_B, _DJ), jnp.float32),
            jax.ShapeDtypeStruct((_B, _DC), jnp.float32),
            jax.ShapeDtypeStruct((_B, _DT), jnp.float32),
        ),
        mesh=mesh,
        compiler_params=pltpu.CompilerParams(use_tc_tiling_on_sc=False),
        scratch_types=[
            pltpu.VMEM((_BPW,), jnp.int32),
            pltpu.VMEM((_BPW,), jnp.int32),
            pltpu.VMEM((_BPW,), jnp.int32),
            pltpu.VMEM((_BPW, _DJ), jnp.float32),
            pltpu.VMEM((_BPW, _DC), jnp.float32),
            pltpu.VMEM((_BPW, _DT), jnp.float32),
            pltpu.SemaphoreType.DMA,
            pltpu.SemaphoreType.DMA,
            pltpu.SemaphoreType.DMA,
        ],
    )
    def gather_kernel(jid_hbm, cid_hbm, tid_hbm, ej_hbm, ec_hbm, et_hbm,
                      oj_hbm, oc_hbm, ot_hbm,
                      ij_v, ic_v, it_v, bj_v, bc_v, bt_v,
                      sem_j, sem_c, sem_t):
        wid = lax.axis_index("s") * _NC + lax.axis_index("c")
        base = wid * _BPW
        pltpu.sync_copy(jid_hbm.at[pl.ds(base, _BPW)], ij_v)
        pltpu.sync_copy(cid_hbm.at[pl.ds(base, _BPW)], ic_v)
        pltpu.sync_copy(tid_hbm.at[pl.ds(base, _BPW)], it_v)
        cp_j = pltpu.async_copy(ej_hbm.at[ij_v], bj_v, sem_j)
        cp_c = pltpu.async_copy(ec_hbm.at[ic_v], bc_v, sem_c)
        cp_t = pltpu.async_copy(et_hbm.at[it_v], bt_v, sem_t)
        cp_j.wait()
        pltpu.sync_copy(bj_v, oj_hbm.at[pl.ds(base, _BPW)])
        cp_c.wait()
        pltpu.sync_copy(bc_v, oc_hbm.at[pl.ds(base, _BPW)])
        cp_t.wait()
        pltpu.sync_copy(bt_v, ot_hbm.at[pl.ds(base, _BPW)])

    return gather_kernel(job_id, company_id, title_id,
                         emb_job, emb_company, emb_title)


_BB = 512
_EPS = float(jnp.finfo(jnp.float32).eps)


def _tc_body(ej_ref, ec_ref, et_ref, df_ref,
             wj_ref, wc_ref, wt_ref, wd_ref, b_ref, o_ref):
    ej = ej_ref[...]
    ec = ec_ref[...]
    et = et_ref[...]
    df = df_ref[...]
    acc = jnp.dot(ej, wj_ref[...], preferred_element_type=jnp.float32)
    acc = acc + jnp.dot(ec, wc_ref[...], preferred_element_type=jnp.float32)
    acc = acc + jnp.dot(et, wt_ref[...], preferred_element_type=jnp.float32)
    acc = acc + jnp.dot(df, wd_ref[...], preferred_element_type=jnp.float32)
    ssq = (jnp.sum(ej * ej, axis=1, keepdims=True)
           + jnp.sum(ec * ec, axis=1, keepdims=True)
           + jnp.sum(et * et, axis=1, keepdims=True)
           + jnp.sum(df * df, axis=1, keepdims=True))
    inv_rms = lax.rsqrt(ssq * (1.0 / _TOTAL) + _EPS)
    o_ref[...] = acc * inv_rms + b_ref[...]


def _tc_fuse(e_job, e_comp, e_title, dense_feats, w_eff, b):
    wj = w_eff[:_DJ]
    wc = w_eff[_DJ:_DJ + _DC]
    wt = w_eff[_DJ + _DC:_DJ + _DC + _DT]
    wd = w_eff[_DJ + _DC + _DT:]
    full = lambda i: (0, 0)
    blk = lambda i: (i, 0)
    return pl.pallas_call(
        _tc_body,
        grid=(_B // _BB,),
        in_specs=[
            pl.BlockSpec((_BB, _DJ), blk),
            pl.BlockSpec((_BB, _DC), blk),
            pl.BlockSpec((_BB, _DT), blk),
            pl.BlockSpec((_BB, _DD), blk),
            pl.BlockSpec((_DJ, _OUT), full),
            pl.BlockSpec((_DC, _OUT), full),
            pl.BlockSpec((_DT, _OUT), full),
            pl.BlockSpec((_DD, _OUT), full),
            pl.BlockSpec((1, _OUT), full),
        ],
        out_specs=pl.BlockSpec((_BB, _OUT), blk),
        out_shape=jax.ShapeDtypeStruct((_B, _OUT), jnp.float32),
    )(e_job, e_comp, e_title, dense_feats, wj, wc, wt, wd,
      b.reshape(1, _OUT))


def kernel(job_id, company_id, title_id, dense_feats, emb_job, emb_company,
           emb_title, rms_scale, W, b):
    job_id = job_id.astype(jnp.int32)
    company_id = company_id.astype(jnp.int32)
    title_id = title_id.astype(jnp.int32)
    e_job, e_comp, e_title = _sc_gather(
        job_id, company_id, title_id, emb_job, emb_company, emb_title)
    w_eff = (W * rms_scale[None, :]).T
    return _tc_fuse(e_job, e_comp, e_title, dense_feats, w_eff, b)


# speedup vs baseline: 1.6107x; 1.6107x over previous
import functools

import jax
import jax.numpy as jnp
from jax import lax
from jax.experimental import pallas as pl
from jax.experimental.pallas import tpu as pltpu
from jax.experimental.pallas import tpu_sc as plsc

_B = 4096
_DJ, _DC, _DT, _DD = 64, 64, 32, 128
_TOTAL = _DJ + _DC + _DT + _DD
_OUT = 128
_NC, _NS = 2, 16
_NW = _NC * _NS
_BPW = _B // _NW
_L = 16
_NCHUNK = _BPW // _L


def _sc_gather(job_id, company_id, title_id, emb_job, emb_company, emb_title):
    mesh = plsc.VectorSubcoreMesh(
        core_axis_name="c", subcore_axis_name="s",
        num_cores=_NC, num_subcores=_NS,
    )

    @functools.partial(
        pl.kernel,
        out_type=(
            jax.ShapeDtypeStruct((_B, _DJ), jnp.float32),
            jax.ShapeDtypeStruct((_B, _DC), jnp.float32),
            jax.ShapeDtypeStruct((_B, _DT), jnp.float32),
        ),
        mesh=mesh,
        compiler_params=pltpu.CompilerParams(needs_layout_passes=False),
        scratch_types=[
            pltpu.VMEM((_BPW,), jnp.int32),
            pltpu.VMEM((_BPW,), jnp.int32),
            pltpu.VMEM((_BPW,), jnp.int32),
            pltpu.VMEM((_BPW, _DJ), jnp.float32),
            pltpu.VMEM((_BPW, _DC), jnp.float32),
            pltpu.VMEM((_BPW, _DT), jnp.float32),
            pltpu.SemaphoreType.DMA,
            pltpu.SemaphoreType.DMA,
            pltpu.SemaphoreType.DMA,
            pltpu.SemaphoreType.DMA,
        ],
    )
    def gather_kernel(jid_hbm, cid_hbm, tid_hbm, ej_hbm, ec_hbm, et_hbm,
                      oj_hbm, oc_hbm, ot_hbm,
                      ij_v, ic_v, it_v, bj_v, bc_v, bt_v,
                      sem_j, sem_c, sem_t, sem_idx):
        wid = lax.axis_index("s") * _NC + lax.axis_index("c")
        base = wid * _BPW
        pltpu.async_copy(jid_hbm.at[pl.ds(base, _BPW)], ij_v, sem_idx).wait()
        pltpu.async_copy(cid_hbm.at[pl.ds(base, _BPW)], ic_v, sem_idx).wait()
        pltpu.async_copy(tid_hbm.at[pl.ds(base, _BPW)], it_v, sem_idx).wait()

        lanes = lax.iota(jnp.int32, _L)

        def fire(g):
            jv = ij_v[pl.ds(g * _L, _L)]
            cv = ic_v[pl.ds(g * _L, _L)]
            tv = it_v[pl.ds(g * _L, _L)]
            for l in range(_L):
                i = g * _L + l
                m = lanes == l
                sj = jnp.sum(jnp.where(m, jv, 0))
                sc = jnp.sum(jnp.where(m, cv, 0))
                st = jnp.sum(jnp.where(m, tv, 0))
                pltpu.async_copy(ej_hbm.at[pl.ds(sj, 1)],
                                 bj_v.at[pl.ds(i, 1)], sem_j)
                pltpu.async_copy(ec_hbm.at[pl.ds(sc, 1)],
                                 bc_v.at[pl.ds(i, 1)], sem_c)
                pltpu.async_copy(et_hbm.at[pl.ds(st, 1)],
                                 bt_v.at[pl.ds(i, 1)], sem_t)

        def drain(g):
            lo = g * _L
            pltpu.make_async_copy(
                ej_hbm.at[pl.ds(0, _L)], bj_v.at[pl.ds(lo, _L)], sem_j).wait()
            pltpu.make_async_copy(
                ec_hbm.at[pl.ds(0, _L)], bc_v.at[pl.ds(lo, _L)], sem_c).wait()
            pltpu.make_async_copy(
                et_hbm.at[pl.ds(0, _L)], bt_v.at[pl.ds(lo, _L)], sem_t).wait()

        fire(0)
        for g in range(1, _NCHUNK):
            fire(g)
            drain(g - 1)
        drain(_NCHUNK - 1)

        pltpu.sync_copy(bj_v, oj_hbm.at[pl.ds(base, _BPW)])
        pltpu.sync_copy(bc_v, oc_hbm.at[pl.ds(base, _BPW)])
        pltpu.sync_copy(bt_v, ot_hbm.at[pl.ds(base, _BPW)])

    return gather_kernel(job_id, company_id, title_id,
                         emb_job, emb_company, emb_title)


_BB = 512
_EPS = float(jnp.finfo(jnp.float32).eps)


def _tc_body(ej_ref, ec_ref, et_ref, df_ref,
             wj_ref, wc_ref, wt_ref, wd_ref, b_ref, o_ref):
    ej = ej_ref[...]
    ec = ec_ref[...]
    et = et_ref[...]
    df = df_ref[...]
    acc = jnp.dot(ej, wj_ref[...], preferred_element_type=jnp.float32)
    acc = acc + jnp.dot(ec, wc_ref[...], preferred_element_type=jnp.float32)
    acc = acc + jnp.dot(et, wt_ref[...], preferred_element_type=jnp.float32)
    acc = acc + jnp.dot(df, wd_ref[...], preferred_element_type=jnp.float32)
    ssq = (jnp.sum(ej * ej, axis=1, keepdims=True)
           + jnp.sum(ec * ec, axis=1, keepdims=True)
           + jnp.sum(et * et, axis=1, keepdims=True)
           + jnp.sum(df * df, axis=1, keepdims=True))
    inv_rms = lax.rsqrt(ssq * (1.0 / _TOTAL) + _EPS)
    o_ref[...] = acc * inv_rms + b_ref[...]


def _tc_fuse(e_job, e_comp, e_title, dense_feats, w_eff, b):
    wj = w_eff[:_DJ]
    wc = w_eff[_DJ:_DJ + _DC]
    wt = w_eff[_DJ + _DC:_DJ + _DC + _DT]
    wd = w_eff[_DJ + _DC + _DT:]
    full = lambda i: (0, 0)
    blk = lambda i: (i, 0)
    return pl.pallas_call(
        _tc_body,
        grid=(_B // _BB,),
        in_specs=[
            pl.BlockSpec((_BB, _DJ), blk),
            pl.BlockSpec((_BB, _DC), blk),
            pl.BlockSpec((_BB, _DT), blk),
            pl.BlockSpec((_BB, _DD), blk),
            pl.BlockSpec((_DJ, _OUT), full),
            pl.BlockSpec((_DC, _OUT), full),
            pl.BlockSpec((_DT, _OUT), full),
            pl.BlockSpec((_DD, _OUT), full),
            pl.BlockSpec((1, _OUT), full),
        ],
        out_specs=pl.BlockSpec((_BB, _OUT), blk),
        out_shape=jax.ShapeDtypeStruct((_B, _OUT), jnp.float32),
    )(e_job, e_comp, e_title, dense_feats, wj, wc, wt, wd,
      b.reshape(1, _OUT))


def kernel(job_id, company_id, title_id, dense_feats, emb_job, emb_company,
           emb_title, rms_scale, W, b):
    job_id = job_id.astype(jnp.int32)
    company_id = company_id.astype(jnp.int32)
    title_id = title_id.astype(jnp.int32)
    e_job, e_comp, e_title = _sc_gather(
        job_id, company_id, title_id, emb_job, emb_company, emb_title)
    w_eff = (W * rms_scale[None, :]).T
    return _tc_fuse(e_job, e_comp, e_title, dense_feats, w_eff, b)
